# baseline (speedup 1.0000x reference)
import functools

import jax
import jax.numpy as jnp
from jax import lax
from jax.experimental import pallas as pl
from jax.experimental.pallas import tpu as pltpu

_N = 10000
_E = 320000
_D = 128
_DE = 16
_HID = 128

_BE = 2560


def _node_mm_body(x_ref, w_ref, o_ref):
    o_ref[...] = jnp.dot(x_ref[...], w_ref[...],
                         preferred_element_type=jnp.float32)


def _node_matmul(x, w):
    n, d = x.shape
    return pl.pallas_call(
        _node_mm_body,
        out_shape=jax.ShapeDtypeStruct((n, w.shape[1]), jnp.float32),
    )(x, w)


def _layer0_body(xs_ref, ea_ref, wie_ref, bi_ref, wh_ref, h0_ref, g0_ref):
    h0 = xs_ref[...] + jnp.dot(ea_ref[...], wie_ref[...],
                               preferred_element_type=jnp.float32) + bi_ref[...]
    h0_ref[...] = h0
    g0_ref[...] = jnp.dot(jnp.maximum(h0, 0.0), wh_ref[...],
                          preferred_element_type=jnp.float32)


def _layer0(xwi_src, edge_attr, wie, bi, wh):
    grid = (_E // _BE,)
    return pl.pallas_call(
        _layer0_body,
        grid=grid,
        in_specs=[
            pl.BlockSpec((_BE, _D), lambda i: (i, 0)),
            pl.BlockSpec((_BE, _DE), lambda i: (i, 0)),
            pl.BlockSpec((_DE, _HID), lambda i: (0, 0)),
            pl.BlockSpec((1, _HID), lambda i: (0, 0)),
            pl.BlockSpec((_HID, _HID), lambda i: (0, 0)),
        ],
        out_specs=[
            pl.BlockSpec((_BE, _HID), lambda i: (i, 0)),
            pl.BlockSpec((_BE, _HID), lambda i: (i, 0)),
        ],
        out_shape=[
            jax.ShapeDtypeStruct((_E, _HID), jnp.float32),
            jax.ShapeDtypeStruct((_E, _HID), jnp.float32),
        ],
        compiler_params=pltpu.CompilerParams(
            dimension_semantics=("parallel",)),
    )(xwi_src, edge_attr, wie, bi, wh)


def _layer_mid_body(h0_ref, asrc_ref, grev_ref, bh_ref, wh_ref, g_ref):
    ht = jnp.maximum(h0_ref[...] + asrc_ref[...] - grev_ref[...]
                     + bh_ref[...], 0.0)
    g_ref[...] = jnp.dot(ht, wh_ref[...], preferred_element_type=jnp.float32)


def _layer_mid(h0, asrc, grev, bh, wh):
    grid = (_E // _BE,)
    return pl.pallas_call(
        _layer_mid_body,
        grid=grid,
        in_specs=[
            pl.BlockSpec((_BE, _HID), lambda i: (i, 0)),
            pl.BlockSpec((_BE, _HID), lambda i: (i, 0)),
            pl.BlockSpec((_BE, _HID), lambda i: (i, 0)),
            pl.BlockSpec((1, _HID), lambda i: (0, 0)),
            pl.BlockSpec((_HID, _HID), lambda i: (0, 0)),
        ],
        out_specs=pl.BlockSpec((_BE, _HID), lambda i: (i, 0)),
        out_shape=jax.ShapeDtypeStruct((_E, _HID), jnp.float32),
        compiler_params=pltpu.CompilerParams(
            dimension_semantics=("parallel",)),
    )(h0, asrc, grev, bh, wh)


def _layer_last_body(h0_ref, asrc_ref, grev_ref, bh_ref, ht_ref):
    ht_ref[...] = jnp.maximum(h0_ref[...] + asrc_ref[...] - grev_ref[...]
                              + bh_ref[...], 0.0)


def _layer_last(h0, asrc, grev, bh):
    grid = (_E // _BE,)
    return pl.pallas_call(
        _layer_last_body,
        grid=grid,
        in_specs=[
            pl.BlockSpec((_BE, _HID), lambda i: (i, 0)),
            pl.BlockSpec((_BE, _HID), lambda i: (i, 0)),
            pl.BlockSpec((_BE, _HID), lambda i: (i, 0)),
            pl.BlockSpec((1, _HID), lambda i: (0, 0)),
        ],
        out_specs=pl.BlockSpec((_BE, _HID), lambda i: (i, 0)),
        out_shape=jax.ShapeDtypeStruct((_E, _HID), jnp.float32),
        compiler_params=pltpu.CompilerParams(
            dimension_semantics=("parallel",)),
    )(h0, asrc, grev, bh)


def _final_body(x_ref, a_ref, wox_ref, wom_ref, bo_ref, o_ref):
    a = a_ref[...]
    rs = jnp.sum(a, axis=1, keepdims=True)
    m = jnp.where(rs == 0.0, x_ref[...], a)
    o = (jnp.dot(x_ref[...], wox_ref[...], preferred_element_type=jnp.float32)
         + jnp.dot(m, wom_ref[...], preferred_element_type=jnp.float32)
         + bo_ref[...])
    o_ref[...] = jnp.maximum(o, 0.0)


def _final(x, a2, wox, wom, bo):
    return pl.pallas_call(
        _final_body,
        out_shape=jax.ShapeDtypeStruct((_N, _HID), jnp.float32),
    )(x, a2, wox, wom, bo.reshape(1, _HID))


def kernel(x, edge_index, edge_attr, rev_edge_index, Wi, bi, Wh, bh, Wo, bo):
    src = edge_index[0]
    dst = edge_index[1]

    xwi = _node_matmul(x, Wi[:_D])
    xwi_src = jnp.take(xwi, src, axis=0)
    h0, g0 = _layer0(xwi_src, edge_attr, Wi[_D:], bi.reshape(1, _HID), Wh)

    a0 = jax.ops.segment_sum(g0, dst, num_segments=_N)
    m_src = jnp.take(a0, src, axis=0)
    g_rev = jnp.take(g0, rev_edge_index, axis=0)
    g1 = _layer_mid(h0, m_src, g_rev, bh.reshape(1, _HID), Wh)

    a1 = jax.ops.segment_sum(g1, dst, num_segments=_N)
    m_src = jnp.take(a1, src, axis=0)
    g_rev = jnp.take(g1, rev_edge_index, axis=0)
    ht2 = _layer_last(h0, m_src, g_rev, bh.reshape(1, _HID))

    a2 = jax.ops.segment_sum(ht2, dst, num_segments=_N)
    return _final(x, a2, Wo[:_D], Wo[_D:], bo)


# speedup vs baseline: 1.6198x; 1.6198x over previous
import functools

import jax
import jax.numpy as jnp
from jax import lax
from jax.experimental import pallas as pl
from jax.experimental.pallas import tpu as pltpu
from jax.experimental.pallas import tpu_sc as plsc

_N = 10000
_E = 320000
_D = 128
_DE = 16
_HID = 128

_BE = 2560

_NC = 2
_NS = 16
_NW = _NC * _NS
_PW = _E // _NW
_GB = 80
_NB = _PW // _GB
_NP = 10240
_NROWS = _NP // _NS


def _sc_mesh():
    return plsc.VectorSubcoreMesh(core_axis_name="c", subcore_axis_name="s")


def _sc_gather_body(table, idx, out, idx_v, rows_v, sem):
    wid = lax.axis_index("s") * _NC + lax.axis_index("c")
    base = wid * _PW

    def step(i, carry):
        off = base + i * _GB
        pltpu.sync_copy(idx.at[pl.ds(off, _GB)], idx_v)
        pltpu.async_copy(table.at[idx_v], rows_v, sem).wait()
        pltpu.sync_copy(rows_v, out.at[pl.ds(off, _GB)])
        return carry

    lax.fori_loop(0, _NB, step, 0)


def _sc_gather(table, idx):
    f = pl.kernel(
        _sc_gather_body,
        out_type=jax.ShapeDtypeStruct((_E, _HID), jnp.float32),
        mesh=_sc_mesh(),
        scratch_types=[
            pltpu.VMEM((_GB,), jnp.int32),
            pltpu.VMEM((_GB, _HID), jnp.float32),
            pltpu.SemaphoreType.DMA,
        ],
    )
    return f(table, idx)


_PW1 = _E // _NS
_NB1 = _PW1 // _GB


def _sc_scatter_body(vals, dst, zeros, out, idx_v, vals_v, acc, sem):
    sid = lax.axis_index("s")

    srow = sid * _NROWS
    pltpu.sync_copy(zeros.at[pl.ds(0, _NROWS)], acc.at[pl.ds(srow, _NROWS)])
    plsc.subcore_barrier()

    base = sid * _PW1

    def step(i, carry):
        off = base + i * _GB
        pltpu.sync_copy(dst.at[pl.ds(off, _GB)], idx_v)
        pltpu.sync_copy(vals.at[pl.ds(off, _GB)], vals_v)
        pltpu.sync_copy(vals_v, acc.at[idx_v], add=True)
        return carry

    lax.fori_loop(0, _NB1, step, 0)
    plsc.subcore_barrier()

    pltpu.sync_copy(acc.at[pl.ds(srow, _NROWS)], out.at[pl.ds(srow, _NROWS)])


def _sc_scatter(vals, dst, zeros):
    f = pl.kernel(
        _sc_scatter_body,
        out_type=jax.ShapeDtypeStruct((_NP, _HID), jnp.float32),
        mesh=plsc.VectorSubcoreMesh(core_axis_name="c", subcore_axis_name="s",
                                    num_cores=1),
        scratch_types=[
            pltpu.VMEM((_GB,), jnp.int32),
            pltpu.VMEM((_GB, _HID), jnp.float32),
            pltpu.VMEM_SHARED((_NP, _HID), jnp.float32),
            pltpu.SemaphoreType.DMA,
        ],
    )
    return f(vals, dst, zeros)


def _combine_body(p_ref, o_ref):
    o_ref[...] = p_ref[0] + p_ref[1]


def _combine(p):
    return pl.pallas_call(
        _combine_body,
        out_shape=jax.ShapeDtypeStruct((_NP, _HID), jnp.float32),
    )(p)


def _node_mm_body(x_ref, w_ref, o_ref):
    o_ref[...] = jnp.dot(x_ref[...], w_ref[...],
                         preferred_element_type=jnp.float32)


def _node_matmul(x, w):
    n, d = x.shape
    return pl.pallas_call(
        _node_mm_body,
        out_shape=jax.ShapeDtypeStruct((n, w.shape[1]), jnp.float32),
    )(x, w)


def _layer0_body(xs_ref, ea_ref, wie_ref, bi_ref, wh_ref, h0_ref, g0_ref):
    h0 = xs_ref[...] + jnp.dot(ea_ref[...], wie_ref[...],
                               preferred_element_type=jnp.float32) + bi_ref[...]
    h0_ref[...] = h0
    g0_ref[...] = jnp.dot(jnp.maximum(h0, 0.0), wh_ref[...],
                          preferred_element_type=jnp.float32)


def _layer0(xwi_src, edge_attr, wie, bi, wh):
    grid = (_E // _BE,)
    return pl.pallas_call(
        _layer0_body,
        grid=grid,
        in_specs=[
            pl.BlockSpec((_BE, _D), lambda i: (i, 0)),
            pl.BlockSpec((_BE, _DE), lambda i: (i, 0)),
            pl.BlockSpec((_DE, _HID), lambda i: (0, 0)),
            pl.BlockSpec((1, _HID), lambda i: (0, 0)),
            pl.BlockSpec((_HID, _HID), lambda i: (0, 0)),
        ],
        out_specs=[
            pl.BlockSpec((_BE, _HID), lambda i: (i, 0)),
            pl.BlockSpec((_BE, _HID), lambda i: (i, 0)),
        ],
        out_shape=[
            jax.ShapeDtypeStruct((_E, _HID), jnp.float32),
            jax.ShapeDtypeStruct((_E, _HID), jnp.float32),
        ],
        compiler_params=pltpu.CompilerParams(
            dimension_semantics=("parallel",)),
    )(xwi_src, edge_attr, wie, bi, wh)


def _layer_mid_body(h0_ref, asrc_ref, grev_ref, bh_ref, wh_ref, g_ref):
    ht = jnp.maximum(h0_ref[...] + asrc_ref[...] - grev_ref[...]
                     + bh_ref[...], 0.0)
    g_ref[...] = jnp.dot(ht, wh_ref[...], preferred_element_type=jnp.float32)


def _layer_mid(h0, asrc, grev, bh, wh):
    grid = (_E // _BE,)
    return pl.pallas_call(
        _layer_mid_body,
        grid=grid,
        in_specs=[
            pl.BlockSpec((_BE, _HID), lambda i: (i, 0)),
            pl.BlockSpec((_BE, _HID), lambda i: (i, 0)),
            pl.BlockSpec((_BE, _HID), lambda i: (i, 0)),
            pl.BlockSpec((1, _HID), lambda i: (0, 0)),
            pl.BlockSpec((_HID, _HID), lambda i: (0, 0)),
        ],
        out_specs=pl.BlockSpec((_BE, _HID), lambda i: (i, 0)),
        out_shape=jax.ShapeDtypeStruct((_E, _HID), jnp.float32),
        compiler_params=pltpu.CompilerParams(
            dimension_semantics=("parallel",)),
    )(h0, asrc, grev, bh, wh)


def _layer_last_body(h0_ref, asrc_ref, grev_ref, bh_ref, ht_ref):
    ht_ref[...] = jnp.maximum(h0_ref[...] + asrc_ref[...] - grev_ref[...]
                              + bh_ref[...], 0.0)


def _layer_last(h0, asrc, grev, bh):
    grid = (_E // _BE,)
    return pl.pallas_call(
        _layer_last_body,
        grid=grid,
        in_specs=[
            pl.BlockSpec((_BE, _HID), lambda i: (i, 0)),
            pl.BlockSpec((_BE, _HID), lambda i: (i, 0)),
            pl.BlockSpec((_BE, _HID), lambda i: (i, 0)),
            pl.BlockSpec((1, _HID), lambda i: (0, 0)),
        ],
        out_specs=pl.BlockSpec((_BE, _HID), lambda i: (i, 0)),
        out_shape=jax.ShapeDtypeStruct((_E, _HID), jnp.float32),
        compiler_params=pltpu.CompilerParams(
            dimension_semantics=("parallel",)),
    )(h0, asrc, grev, bh)


def _final_body(x_ref, a_ref, wox_ref, wom_ref, bo_ref, o_ref):
    a = a_ref[...][:_N]
    rs = jnp.sum(a, axis=1, keepdims=True)
    m = jnp.where(rs == 0.0, x_ref[...], a)
    o = (jnp.dot(x_ref[...], wox_ref[...], preferred_element_type=jnp.float32)
         + jnp.dot(m, wom_ref[...], preferred_element_type=jnp.float32)
         + bo_ref[...])
    o_ref[...] = jnp.maximum(o, 0.0)


def _final(x, a2, wox, wom, bo):
    return pl.pallas_call(
        _final_body,
        out_shape=jax.ShapeDtypeStruct((_N, _HID), jnp.float32),
    )(x, a2, wox, wom, bo.reshape(1, _HID))


def kernel(x, edge_index, edge_attr, rev_edge_index, Wi, bi, Wh, bh, Wo, bo):
    src = edge_index[0]
    dst = edge_index[1]

    zeros = jnp.zeros((_NROWS, _HID), jnp.float32)

    xwi = _node_matmul(x, Wi[:_D])
    xwi_src = _sc_gather(xwi, src)
    h0, g0 = _layer0(xwi_src, edge_attr, Wi[_D:], bi.reshape(1, _HID), Wh)

    a0 = _sc_scatter(g0, dst, zeros)
    m_src = _sc_gather(a0, src)
    g_rev = _sc_gather(g0, rev_edge_index)
    g1 = _layer_mid(h0, m_src, g_rev, bh.reshape(1, _HID), Wh)

    a1 = _sc_scatter(g1, dst, zeros)
    m_src = _sc_gather(a1, src)
    g_rev = _sc_gather(g1, rev_edge_index)
    ht2 = _layer_last(h0, m_src, g_rev, bh.reshape(1, _HID))

    a2 = _sc_scatter(ht2, dst, zeros)
    return _final(x, a2, Wo[:_D], Wo[_D:], bo)


# speedup vs baseline: 2.7860x; 1.7200x over previous
import functools

import jax
import jax.numpy as jnp
from jax import lax
from jax.experimental import pallas as pl
from jax.experimental.pallas import tpu as pltpu
from jax.experimental.pallas import tpu_sc as plsc

_N = 10000
_E = 320000
_D = 128
_DE = 16
_HID = 128
_HC = _HID // 2

_BE = 2560

_NC = 2
_NS = 16
_NW = _NC * _NS
_GB = 80
_KB = 4
_EP = 327680
_PADE = _EP - _E
_NBLK = _EP // _GB
_RW = _NBLK // _NW
_RW1 = _NBLK // _NS
_NP = 10240
_NROWS = _NP // _NS
_DR = 16
_DCH = _NROWS // _DR


def _sc_mesh(num_cores=_NC):
    return plsc.VectorSubcoreMesh(core_axis_name="c", subcore_axis_name="s",
                                  num_cores=num_cores)


def _sc_gather_body(table, idx2, out, idx_v, bufs, sem):
    wid = lax.axis_index("s") * _NC + lax.axis_index("c")
    rbase = wid * _RW
    pltpu.sync_copy(idx2.at[pl.ds(rbase, _RW)], idx_v)

    def burst(g, carry):
        descs = [
            pltpu.async_copy(table.at[idx_v.at[g * _KB + b]], bufs[b], sem)
            for b in range(_KB)
        ]
        for d in descs:
            d.wait()
        obase = (rbase + g * _KB) * _GB
        descs = [
            pltpu.async_copy(bufs[b], out.at[pl.ds(obase + b * _GB, _GB)], sem)
            for b in range(_KB)
        ]
        for d in descs:
            d.wait()
        return carry

    lax.fori_loop(0, _RW // _KB, burst, 0)


def _sc_gather(table, idx2):
    f = pl.kernel(
        _sc_gather_body,
        out_type=jax.ShapeDtypeStruct((_EP, _HID), jnp.float32),
        mesh=_sc_mesh(),
        scratch_types=[
            pltpu.VMEM((_RW, _GB), jnp.int32),
            [pltpu.VMEM((_GB, _HID), jnp.float32) for _ in range(_KB)],
            pltpu.SemaphoreType.DMA,
        ],
    )
    return f(table, idx2)


def _sc_msg_body(a, gn, src2, rev2, out, si_v, ri_v, bufs, sem):
    wid = lax.axis_index("s") * _NC + lax.axis_index("c")
    rbase = wid * _RW
    pltpu.sync_copy(src2.at[pl.ds(rbase, _RW)], si_v)
    pltpu.sync_copy(rev2.at[pl.ds(rbase, _RW)], ri_v)

    def burst(g, carry):
        descs = [
            pltpu.async_copy(a.at[si_v.at[g * _KB + b]], bufs[b], sem)
            for b in range(_KB)
        ]
        for d in descs:
            d.wait()
        descs = [
            pltpu.async_copy(gn.at[ri_v.at[g * _KB + b]], bufs[b], sem,
                             add=True)
            for b in range(_KB)
        ]
        for d in descs:
            d.wait()
        obase = (rbase + g * _KB) * _GB
        descs = [
            pltpu.async_copy(bufs[b], out.at[pl.ds(obase + b * _GB, _GB)], sem)
            for b in range(_KB)
        ]
        for d in descs:
            d.wait()
        return carry

    lax.fori_loop(0, _RW // _KB, burst, 0)


def _sc_msg(a, gn, src2, rev2):
    f = pl.kernel(
        _sc_msg_body,
        out_type=jax.ShapeDtypeStruct((_EP, _HID), jnp.float32),
        mesh=_sc_mesh(),
        scratch_types=[
            pltpu.VMEM((_RW, _GB), jnp.int32),
            pltpu.VMEM((_RW, _GB), jnp.int32),
            [pltpu.VMEM((_GB, _HID), jnp.float32) for _ in range(_KB)],
            pltpu.SemaphoreType.DMA,
        ],
    )
    return f(a, gn, src2, rev2)


_PW1 = _EP // _NS
_NB1 = _PW1 // _GB


def _sc_scatter_common(vals, dst1, zeros, out, idx_bufs, bufs, stage_v, acc,
                       sem, isem, negate_out):
    sid = lax.axis_index("s")

    srow = sid * _NROWS
    pltpu.sync_copy(zeros.at[pl.ds(0, _NROWS)], acc.at[pl.ds(srow, _NROWS)])
    plsc.subcore_barrier()

    base = sid * _PW1

    def burst(g, carry):
        obase = base + g * _KB * _GB
        idescs = [
            pltpu.async_copy(dst1.at[pl.ds(obase + b * _GB, _GB)],
                             idx_bufs[b], isem)
            for b in range(_KB)
        ]
        vdescs = [
            pltpu.async_copy(vals.at[pl.ds(obase + b * _GB, _GB)], bufs[b],
                             sem)
            for b in range(_KB)
        ]
        for d in idescs:
            d.wait()
        for d in vdescs:
            d.wait()
        descs = [
            pltpu.async_copy(bufs[b], acc.at[idx_bufs[b]], sem, add=True)
            for b in range(_KB)
        ]
        for d in descs:
            d.wait()
        return carry

    lax.fori_loop(0, _NB1 // _KB, burst, 0)
    plsc.subcore_barrier()

    if not negate_out:
        pltpu.sync_copy(acc.at[pl.ds(srow, _NROWS)],
                        out.at[pl.ds(srow, _NROWS)])
    else:
        def dump(c, carry):
            row = srow + c * _DR
            pltpu.sync_copy(acc.at[pl.ds(row, _DR)], stage_v)

            def negrow(r, carry2):
                for cc in range(_HID // 16):
                    sl = pl.ds(cc * 16, 16)
                    stage_v[r, sl] = -stage_v[r, sl]
                return carry2

            lax.fori_loop(0, _DR, negrow, 0)
            pltpu.sync_copy(stage_v, out.at[pl.ds(row, _DR)])
            return carry

        lax.fori_loop(0, _DCH, dump, 0)


def _sc_scatter_body(vals, dst1, zeros, out, idx_bufs, bufs, stage_v, acc,
                     sem, isem):
    _sc_scatter_common(vals, dst1, zeros, out, idx_bufs, bufs, stage_v, acc,
                       sem, isem, negate_out=False)


def _sc_scatter_neg_body(vals, dst1, zeros, out, idx_bufs, bufs, stage_v, acc,
                         sem, isem):
    _sc_scatter_common(vals, dst1, zeros, out, idx_bufs, bufs, stage_v, acc,
                       sem, isem, negate_out=True)


def _sc_scatter(vals, dst1, zeros, negate_out=False):
    f = pl.kernel(
        _sc_scatter_neg_body if negate_out else _sc_scatter_body,
        out_type=jax.ShapeDtypeStruct((_NP, _HID), jnp.float32),
        mesh=_sc_mesh(num_cores=1),
        scratch_types=[
            [pltpu.VMEM((_GB,), jnp.int32) for _ in range(_KB)],
            [pltpu.VMEM((_GB, _HID), jnp.float32) for _ in range(_KB)],
            pltpu.VMEM((_DR, _HID), jnp.float32),
            pltpu.VMEM_SHARED((_NP, _HID), jnp.float32),
            pltpu.SemaphoreType.DMA,
            pltpu.SemaphoreType.DMA,
        ],
    )
    return f(vals, dst1, zeros)


def _node_mm_body(x_ref, w_ref, o_ref):
    o_ref[...] = jnp.dot(x_ref[...], w_ref[...],
                         preferred_element_type=jnp.float32)


def _node_matmul(x, w):
    n, d = x.shape
    return pl.pallas_call(
        _node_mm_body,
        out_shape=jax.ShapeDtypeStruct((n, w.shape[1]), jnp.float32),
    )(x, w)


def _layer0_body(xs_ref, ea_ref, wie_ref, bi_ref, bh_ref, whn_ref,
                 h0b_ref, g0n_ref):
    h0 = xs_ref[...] + jnp.dot(ea_ref[...], wie_ref[...],
                               preferred_element_type=jnp.float32) + bi_ref[...]
    h0b_ref[...] = h0 + bh_ref[...]
    g0n_ref[...] = jnp.dot(jnp.maximum(h0, 0.0), whn_ref[...],
                           preferred_element_type=jnp.float32)


def _layer0(xwi_src, edge_attr, wie, bi, bh, whn):
    grid = (_E // _BE,)
    return pl.pallas_call(
        _layer0_body,
        grid=grid,
        in_specs=[
            pl.BlockSpec((_BE, _D), lambda i: (i, 0)),
            pl.BlockSpec((_BE, _DE), lambda i: (i, 0)),
            pl.BlockSpec((_DE, _HID), lambda i: (0, 0)),
            pl.BlockSpec((1, _HID), lambda i: (0, 0)),
            pl.BlockSpec((1, _HID), lambda i: (0, 0)),
            pl.BlockSpec((_HID, _HID), lambda i: (0, 0)),
        ],
        out_specs=[
            pl.BlockSpec((_BE, _HID), lambda i: (i, 0)),
            pl.BlockSpec((_BE, _HID), lambda i: (i, 0)),
        ],
        out_shape=[
            jax.ShapeDtypeStruct((_EP, _HID), jnp.float32),
            jax.ShapeDtypeStruct((_EP, _HID), jnp.float32),
        ],
        compiler_params=pltpu.CompilerParams(
            dimension_semantics=("parallel",)),
    )(xwi_src, edge_attr, wie, bi, bh, whn)


def _layer_mid_body(h0b_ref, m_ref, whn_ref, g_ref):
    ht = jnp.maximum(h0b_ref[...] + m_ref[...], 0.0)
    g_ref[...] = jnp.dot(ht, whn_ref[...], preferred_element_type=jnp.float32)


def _layer_mid(h0b, m, whn):
    grid = (_E // _BE,)
    return pl.pallas_call(
        _layer_mid_body,
        grid=grid,
        in_specs=[
            pl.BlockSpec((_BE, _HID), lambda i: (i, 0)),
            pl.BlockSpec((_BE, _HID), lambda i: (i, 0)),
            pl.BlockSpec((_HID, _HID), lambda i: (0, 0)),
        ],
        out_specs=pl.BlockSpec((_BE, _HID), lambda i: (i, 0)),
        out_shape=jax.ShapeDtypeStruct((_EP, _HID), jnp.float32),
        compiler_params=pltpu.CompilerParams(
            dimension_semantics=("parallel",)),
    )(h0b, m, whn)


def _layer_last_body(h0b_ref, m_ref, ht_ref):
    ht_ref[...] = jnp.maximum(h0b_ref[...] + m_ref[...], 0.0)


def _layer_last(h0b, m):
    grid = (_E // _BE,)
    return pl.pallas_call(
        _layer_last_body,
        grid=grid,
        in_specs=[
            pl.BlockSpec((_BE, _HID), lambda i: (i, 0)),
            pl.BlockSpec((_BE, _HID), lambda i: (i, 0)),
        ],
        out_specs=pl.BlockSpec((_BE, _HID), lambda i: (i, 0)),
        out_shape=jax.ShapeDtypeStruct((_EP, _HID), jnp.float32),
        compiler_params=pltpu.CompilerParams(
            dimension_semantics=("parallel",)),
    )(h0b, m)


def _final_body(x_ref, a_ref, wox_ref, wom_ref, bo_ref, o_ref):
    a = a_ref[...][:_N]
    rs = jnp.sum(a, axis=1, keepdims=True)
    m = jnp.where(rs == 0.0, x_ref[...], a)
    o = (jnp.dot(x_ref[...], wox_ref[...], preferred_element_type=jnp.float32)
         + jnp.dot(m, wom_ref[...], preferred_element_type=jnp.float32)
         + bo_ref[...])
    o_ref[...] = jnp.maximum(o, 0.0)


def _final(x, a2, wox, wom, bo):
    return pl.pallas_call(
        _final_body,
        out_shape=jax.ShapeDtypeStruct((_N, _HID), jnp.float32),
    )(x, a2, wox, wom, bo.reshape(1, _HID))


def kernel(x, edge_index, edge_attr, rev_edge_index, Wi, bi, Wh, bh, Wo, bo):
    pad = jnp.arange(_PADE, dtype=jnp.int32)
    src2 = jnp.concatenate([edge_index[0], pad % _N]).reshape(_NBLK, _GB)
    dst1 = jnp.concatenate([edge_index[1], _N + pad % (_NP - _N)])
    rev2 = jnp.concatenate([rev_edge_index, pad % _E]).reshape(_NBLK, _GB)

    zeros = jnp.zeros((_NROWS, _HID), jnp.float32)
    whn = -Wh

    xwi = _node_matmul(x, Wi[:_D])
    xwi_src = _sc_gather(xwi, src2)
    h0b, g0n = _layer0(xwi_src, edge_attr, Wi[_D:], bi.reshape(1, _HID),
                       bh.reshape(1, _HID), whn)

    a0 = _sc_scatter(g0n, dst1, zeros, negate_out=True)
    m1 = _sc_msg(a0, g0n, src2, rev2)
    g1n = _layer_mid(h0b, m1, whn)

    a1 = _sc_scatter(g1n, dst1, zeros, negate_out=True)
    m2 = _sc_msg(a1, g1n, src2, rev2)
    ht2 = _layer_last(h0b, m2)

    a2 = _sc_scatter(ht2, dst1, zeros)
    return _final(x, a2, Wo[:_D], Wo[_D:], bo)


# speedup vs baseline: 3.3731x; 1.2107x over previous
import functools

import jax
import jax.numpy as jnp
from jax import lax
from jax.experimental import pallas as pl
from jax.experimental.pallas import tpu as pltpu
from jax.experimental.pallas import tpu_sc as plsc

_N = 10000
_E = 320000
_D = 128
_DE = 16
_HID = 128
_HC = _HID // 2

_BE = 2560

_NC = 2
_NS = 16
_NW = _NC * _NS
_GB = 80
_KB = 4
_EP = 327680
_PADE = _EP - _E
_NBLK = _EP // _GB
_RW = _NBLK // _NW
_RW1 = _NBLK // _NS
_NP = 10240
_NROWS = _NP // _NS
_DR = 16
_DCH = _NROWS // _DR


def _sc_mesh(num_cores=_NC):
    return plsc.VectorSubcoreMesh(core_axis_name="c", subcore_axis_name="s",
                                  num_cores=num_cores)


def _sc_gather_body(table, idx2, out, idx_v, bufs, sem):
    wid = lax.axis_index("s") * _NC + lax.axis_index("c")
    rbase = wid * _RW
    pltpu.sync_copy(idx2.at[pl.ds(rbase, _RW)], idx_v)

    def burst(g, carry):
        descs = [
            pltpu.async_copy(table.at[idx_v.at[g * _KB + b]], bufs[b], sem)
            for b in range(_KB)
        ]
        for d in descs:
            d.wait()
        obase = (rbase + g * _KB) * _GB
        descs = [
            pltpu.async_copy(bufs[b], out.at[pl.ds(obase + b * _GB, _GB)], sem)
            for b in range(_KB)
        ]
        for d in descs:
            d.wait()
        return carry

    lax.fori_loop(0, _RW // _KB, burst, 0)


def _sc_gather(table, idx2):
    f = pl.kernel(
        _sc_gather_body,
        out_type=jax.ShapeDtypeStruct((_EP, _HID), jnp.float32),
        mesh=_sc_mesh(),
        scratch_types=[
            pltpu.VMEM((_RW, _GB), jnp.int32),
            [pltpu.VMEM((_GB, _HID), jnp.float32) for _ in range(_KB)],
            pltpu.SemaphoreType.DMA,
        ],
    )
    return f(table, idx2)


def _sc_msg_body(a, gn, src2, rev2, out, si_v, ri_v, bufs, sem):
    wid = lax.axis_index("s") * _NC + lax.axis_index("c")
    rbase = wid * _RW
    pltpu.sync_copy(src2.at[pl.ds(rbase, _RW)], si_v)
    pltpu.sync_copy(rev2.at[pl.ds(rbase, _RW)], ri_v)

    def burst(g, carry):
        descs = [
            pltpu.async_copy(a.at[si_v.at[g * _KB + b]], bufs[b], sem)
            for b in range(_KB)
        ]
        for d in descs:
            d.wait()
        descs = [
            pltpu.async_copy(gn.at[ri_v.at[g * _KB + b]], bufs[b], sem,
                             add=True)
            for b in range(_KB)
        ]
        for d in descs:
            d.wait()
        obase = (rbase + g * _KB) * _GB
        descs = [
            pltpu.async_copy(bufs[b], out.at[pl.ds(obase + b * _GB, _GB)], sem)
            for b in range(_KB)
        ]
        for d in descs:
            d.wait()
        return carry

    lax.fori_loop(0, _RW // _KB, burst, 0)


def _sc_msg(a, gn, src2, rev2):
    f = pl.kernel(
        _sc_msg_body,
        out_type=jax.ShapeDtypeStruct((_EP, _HID), jnp.float32),
        mesh=_sc_mesh(),
        scratch_types=[
            pltpu.VMEM((_RW, _GB), jnp.int32),
            pltpu.VMEM((_RW, _GB), jnp.int32),
            [pltpu.VMEM((_GB, _HID), jnp.float32) for _ in range(_KB)],
            pltpu.SemaphoreType.DMA,
        ],
    )
    return f(a, gn, src2, rev2)


_PW1 = _EP // _NW
_NB1 = _PW1 // _GB


def _sc_scatter_body(vals, dst1, zeros, out, idx_bufs, bufs, acc, sem, isem):
    cid = lax.axis_index("c")
    sid = lax.axis_index("s")

    srow = sid * _NROWS
    pltpu.sync_copy(zeros.at[pl.ds(0, _NROWS)], acc.at[pl.ds(srow, _NROWS)])
    plsc.subcore_barrier()

    base = cid * (_EP // _NC) + sid * _PW1

    def burst(g, carry):
        obase = base + g * _KB * _GB
        idescs = [
            pltpu.async_copy(dst1.at[pl.ds(obase + b * _GB, _GB)],
                             idx_bufs[b], isem)
            for b in range(_KB)
        ]
        vdescs = [
            pltpu.async_copy(vals.at[pl.ds(obase + b * _GB, _GB)], bufs[b],
                             sem)
            for b in range(_KB)
        ]
        for d in idescs:
            d.wait()
        for d in vdescs:
            d.wait()
        descs = [
            pltpu.async_copy(bufs[b], acc.at[idx_bufs[b]], sem, add=True)
            for b in range(_KB)
        ]
        for d in descs:
            d.wait()
        return carry

    lax.fori_loop(0, _NB1 // _KB, burst, 0)
    plsc.subcore_barrier()

    pltpu.sync_copy(acc.at[pl.ds(srow, _NROWS)],
                    out.at[cid, pl.ds(srow, _NROWS)])


def _sc_scatter(vals, dst1, zeros):
    f = pl.kernel(
        _sc_scatter_body,
        out_type=jax.ShapeDtypeStruct((_NC, _NP, _HID), jnp.float32),
        mesh=_sc_mesh(),
        scratch_types=[
            [pltpu.VMEM((_GB,), jnp.int32) for _ in range(_KB)],
            [pltpu.VMEM((_GB, _HID), jnp.float32) for _ in range(_KB)],
            pltpu.VMEM_SHARED((_NP, _HID), jnp.float32),
            pltpu.SemaphoreType.DMA,
            pltpu.SemaphoreType.DMA,
        ],
    )
    return f(vals, dst1, zeros)


def _combine_neg_body(p_ref, o_ref):
    o_ref[...] = -(p_ref[0] + p_ref[1])


def _combine_neg(p):
    return pl.pallas_call(
        _combine_neg_body,
        out_shape=jax.ShapeDtypeStruct((_NP, _HID), jnp.float32),
    )(p)


def _node_mm_body(x_ref, w_ref, o_ref):
    o_ref[...] = jnp.dot(x_ref[...], w_ref[...],
                         preferred_element_type=jnp.float32)


def _node_matmul(x, w):
    n, d = x.shape
    return pl.pallas_call(
        _node_mm_body,
        out_shape=jax.ShapeDtypeStruct((n, w.shape[1]), jnp.float32),
    )(x, w)


def _layer0_body(xs_ref, ea_ref, wie_ref, bi_ref, bh_ref, whn_ref,
                 h0b_ref, g0n_ref):
    h0 = xs_ref[...] + jnp.dot(ea_ref[...], wie_ref[...],
                               preferred_element_type=jnp.float32) + bi_ref[...]
    h0b_ref[...] = h0 + bh_ref[...]
    g0n_ref[...] = jnp.dot(jnp.maximum(h0, 0.0), whn_ref[...],
                           preferred_element_type=jnp.float32)


def _layer0(xwi_src, edge_attr, wie, bi, bh, whn):
    grid = (_E // _BE,)
    return pl.pallas_call(
        _layer0_body,
        grid=grid,
        in_specs=[
            pl.BlockSpec((_BE, _D), lambda i: (i, 0)),
            pl.BlockSpec((_BE, _DE), lambda i: (i, 0)),
            pl.BlockSpec((_DE, _HID), lambda i: (0, 0)),
            pl.BlockSpec((1, _HID), lambda i: (0, 0)),
            pl.BlockSpec((1, _HID), lambda i: (0, 0)),
            pl.BlockSpec((_HID, _HID), lambda i: (0, 0)),
        ],
        out_specs=[
            pl.BlockSpec((_BE, _HID), lambda i: (i, 0)),
            pl.BlockSpec((_BE, _HID), lambda i: (i, 0)),
        ],
        out_shape=[
            jax.ShapeDtypeStruct((_EP, _HID), jnp.float32),
            jax.ShapeDtypeStruct((_EP, _HID), jnp.float32),
        ],
        compiler_params=pltpu.CompilerParams(
            dimension_semantics=("parallel",)),
    )(xwi_src, edge_attr, wie, bi, bh, whn)


def _layer_mid_body(h0b_ref, m_ref, whn_ref, g_ref):
    ht = jnp.maximum(h0b_ref[...] + m_ref[...], 0.0)
    g_ref[...] = jnp.dot(ht, whn_ref[...], preferred_element_type=jnp.float32)


def _layer_mid(h0b, m, whn):
    grid = (_E // _BE,)
    return pl.pallas_call(
        _layer_mid_body,
        grid=grid,
        in_specs=[
            pl.BlockSpec((_BE, _HID), lambda i: (i, 0)),
            pl.BlockSpec((_BE, _HID), lambda i: (i, 0)),
            pl.BlockSpec((_HID, _HID), lambda i: (0, 0)),
        ],
        out_specs=pl.BlockSpec((_BE, _HID), lambda i: (i, 0)),
        out_shape=jax.ShapeDtypeStruct((_EP, _HID), jnp.float32),
        compiler_params=pltpu.CompilerParams(
            dimension_semantics=("parallel",)),
    )(h0b, m, whn)


def _layer_last_body(h0b_ref, m_ref, ht_ref):
    ht_ref[...] = jnp.maximum(h0b_ref[...] + m_ref[...], 0.0)


def _layer_last(h0b, m):
    grid = (_E // _BE,)
    return pl.pallas_call(
        _layer_last_body,
        grid=grid,
        in_specs=[
            pl.BlockSpec((_BE, _HID), lambda i: (i, 0)),
            pl.BlockSpec((_BE, _HID), lambda i: (i, 0)),
        ],
        out_specs=pl.BlockSpec((_BE, _HID), lambda i: (i, 0)),
        out_shape=jax.ShapeDtypeStruct((_EP, _HID), jnp.float32),
        compiler_params=pltpu.CompilerParams(
            dimension_semantics=("parallel",)),
    )(h0b, m)


def _final_body(x_ref, a_ref, wox_ref, wom_ref, bo_ref, o_ref):
    a = a_ref[0, :_N] + a_ref[1, :_N]
    rs = jnp.sum(a, axis=1, keepdims=True)
    m = jnp.where(rs == 0.0, x_ref[...], a)
    o = (jnp.dot(x_ref[...], wox_ref[...], preferred_element_type=jnp.float32)
         + jnp.dot(m, wom_ref[...], preferred_element_type=jnp.float32)
         + bo_ref[...])
    o_ref[...] = jnp.maximum(o, 0.0)


def _final(x, a2, wox, wom, bo):
    return pl.pallas_call(
        _final_body,
        out_shape=jax.ShapeDtypeStruct((_N, _HID), jnp.float32),
    )(x, a2, wox, wom, bo.reshape(1, _HID))


def kernel(x, edge_index, edge_attr, rev_edge_index, Wi, bi, Wh, bh, Wo, bo):
    pad = jnp.arange(_PADE, dtype=jnp.int32)
    src2 = jnp.concatenate([edge_index[0], pad % _N]).reshape(_NBLK, _GB)
    dst1 = jnp.concatenate([edge_index[1], _N + pad % (_NP - _N)])
    rev2 = jnp.concatenate([rev_edge_index, pad % _E]).reshape(_NBLK, _GB)

    zeros = jnp.zeros((_NROWS, _HID), jnp.float32)
    whn = -Wh

    xwi = _node_matmul(x, Wi[:_D])
    xwi_src = _sc_gather(xwi, src2)
    h0b, g0n = _layer0(xwi_src, edge_attr, Wi[_D:], bi.reshape(1, _HID),
                       bh.reshape(1, _HID), whn)

    a0 = _combine_neg(_sc_scatter(g0n, dst1, zeros))
    m1 = _sc_msg(a0, g0n, src2, rev2)
    g1n = _layer_mid(h0b, m1, whn)

    a1 = _combine_neg(_sc_scatter(g1n, dst1, zeros))
    m2 = _sc_msg(a1, g1n, src2, rev2)
    ht2 = _layer_last(h0b, m2)

    a2p = _sc_scatter(ht2, dst1, zeros)
    return _final(x, a2p, Wo[:_D], Wo[_D:], bo)


# speedup vs baseline: 3.7853x; 1.1222x over previous
import functools

import jax
import jax.numpy as jnp
from jax import lax
from jax.experimental import pallas as pl
from jax.experimental.pallas import tpu as pltpu
from jax.experimental.pallas import tpu_sc as plsc

_N = 10000
_E = 320000
_D = 128
_DE = 16
_HID = 128
_HC = _HID // 2

_BE = 2560

_NC = 2
_NS = 16
_NW = _NC * _NS
_GB = 80
_KB = 4
_EP = 327680
_PADE = _EP - _E
_NBLK = _EP // _GB
_RW = _NBLK // _NW
_RW1 = _NBLK // _NS
_NP = 10240
_NROWS = _NP // _NS
_DR = 16
_DCH = _NROWS // _DR


def _sc_mesh(num_cores=_NC):
    return plsc.VectorSubcoreMesh(core_axis_name="c", subcore_axis_name="s",
                                  num_cores=num_cores)


def _sc_gather_body(table, idx2, out, idx_v, bufs, sem):
    wid = lax.axis_index("s") * _NC + lax.axis_index("c")
    rbase = wid * _RW
    pltpu.sync_copy(idx2.at[pl.ds(rbase, _RW)], idx_v)

    def burst(g, carry):
        descs = [
            pltpu.async_copy(table.at[idx_v.at[g * _KB + b]], bufs[b], sem)
            for b in range(_KB)
        ]
        for d in descs:
            d.wait()
        obase = (rbase + g * _KB) * _GB
        descs = [
            pltpu.async_copy(bufs[b], out.at[pl.ds(obase + b * _GB, _GB)], sem)
            for b in range(_KB)
        ]
        for d in descs:
            d.wait()
        return carry

    lax.fori_loop(0, _RW // _KB, burst, 0)


def _sc_gather(table, idx2):
    f = pl.kernel(
        _sc_gather_body,
        out_type=jax.ShapeDtypeStruct((_EP, _HID), jnp.float32),
        mesh=_sc_mesh(),
        scratch_types=[
            pltpu.VMEM((_RW, _GB), jnp.int32),
            [pltpu.VMEM((_GB, _HID), jnp.float32) for _ in range(_KB)],
            pltpu.SemaphoreType.DMA,
        ],
    )
    return f(table, idx2)


def _sc_msg_body(a, gn, src2, rev2, out, si_v, ri_v, bufs, sem):
    wid = lax.axis_index("s") * _NC + lax.axis_index("c")
    rbase = wid * _RW
    pltpu.sync_copy(src2.at[pl.ds(rbase, _RW)], si_v)
    pltpu.sync_copy(rev2.at[pl.ds(rbase, _RW)], ri_v)

    def burst(g, carry):
        descs = [
            pltpu.async_copy(a.at[si_v.at[g * _KB + b]], bufs[b], sem)
            for b in range(_KB)
        ]
        for d in descs:
            d.wait()
        descs = [
            pltpu.async_copy(gn.at[ri_v.at[g * _KB + b]], bufs[b], sem,
                             add=True)
            for b in range(_KB)
        ]
        for d in descs:
            d.wait()
        obase = (rbase + g * _KB) * _GB
        descs = [
            pltpu.async_copy(bufs[b], out.at[pl.ds(obase + b * _GB, _GB)], sem)
            for b in range(_KB)
        ]
        for d in descs:
            d.wait()
        return carry

    lax.fori_loop(0, _RW // _KB, burst, 0)


def _sc_msg(a, gn, src2, rev2):
    f = pl.kernel(
        _sc_msg_body,
        out_type=jax.ShapeDtypeStruct((_EP, _HID), jnp.float32),
        mesh=_sc_mesh(),
        scratch_types=[
            pltpu.VMEM((_RW, _GB), jnp.int32),
            pltpu.VMEM((_RW, _GB), jnp.int32),
            [pltpu.VMEM((_GB, _HID), jnp.float32) for _ in range(_KB)],
            pltpu.SemaphoreType.DMA,
        ],
    )
    return f(a, gn, src2, rev2)


_PW1 = _EP // _NW
_NB1 = _PW1 // _GB


def _sc_scatter_body(vals, dst1, zeros, out, idx_bufs, bufs, acc, sem, isem):
    cid = lax.axis_index("c")
    sid = lax.axis_index("s")

    srow = sid * _NROWS
    pltpu.sync_copy(zeros.at[pl.ds(0, _NROWS)], acc.at[pl.ds(srow, _NROWS)])
    plsc.subcore_barrier()

    base = cid * (_EP // _NC) + sid * _PW1

    def burst(g, carry):
        obase = base + g * _KB * _GB
        idescs = [
            pltpu.async_copy(dst1.at[pl.ds(obase + b * _GB, _GB)],
                             idx_bufs[b], isem)
            for b in range(_KB)
        ]
        vdescs = [
            pltpu.async_copy(vals.at[pl.ds(obase + b * _GB, _GB)], bufs[b],
                             sem)
            for b in range(_KB)
        ]
        for d in idescs:
            d.wait()
        for d in vdescs:
            d.wait()
        descs = [
            pltpu.async_copy(bufs[b], acc.at[idx_bufs[b]], sem, add=True)
            for b in range(_KB)
        ]
        for d in descs:
            d.wait()
        return carry

    lax.fori_loop(0, _NB1 // _KB, burst, 0)
    plsc.subcore_barrier()

    pltpu.sync_copy(acc.at[pl.ds(srow, _NROWS)],
                    out.at[cid, pl.ds(srow, _NROWS)])


def _sc_scatter(vals, dst1, zeros):
    f = pl.kernel(
        _sc_scatter_body,
        out_type=jax.ShapeDtypeStruct((_NC, _NP, _HID), jnp.float32),
        mesh=_sc_mesh(),
        scratch_types=[
            [pltpu.VMEM((_GB,), jnp.int32) for _ in range(_KB)],
            [pltpu.VMEM((_GB, _HID), jnp.float32) for _ in range(_KB)],
            pltpu.VMEM_SHARED((_NP, _HID), jnp.float32),
            pltpu.SemaphoreType.DMA,
            pltpu.SemaphoreType.DMA,
        ],
    )
    return f(vals, dst1, zeros)


def _sc_fin_body(h0b, a, gn, src1, rev1, dst1, zeros, out,
                 si_bufs, ri_bufs, di_bufs, bufs, acc, sem, isem):
    cid = lax.axis_index("c")
    sid = lax.axis_index("s")

    srow = sid * _NROWS
    pltpu.sync_copy(zeros.at[pl.ds(0, _NROWS)], acc.at[pl.ds(srow, _NROWS)])
    plsc.subcore_barrier()

    base = cid * (_EP // _NC) + sid * _PW1

    def burst(g, carry):
        obase = base + g * _KB * _GB
        idescs = []
        for b in range(_KB):
            off = pl.ds(obase + b * _GB, _GB)
            idescs.append(pltpu.async_copy(src1.at[off], si_bufs[b], isem))
            idescs.append(pltpu.async_copy(rev1.at[off], ri_bufs[b], isem))
            idescs.append(pltpu.async_copy(dst1.at[off], di_bufs[b], isem))
        hdescs = [
            pltpu.async_copy(h0b.at[pl.ds(obase + b * _GB, _GB)], bufs[b],
                             sem)
            for b in range(_KB)
        ]
        for d in idescs:
            d.wait()
        for d in hdescs:
            d.wait()
        descs = [
            pltpu.async_copy(a.at[si_bufs[b]], bufs[b], sem, add=True)
            for b in range(_KB)
        ]
        for d in descs:
            d.wait()
        descs = [
            pltpu.async_copy(gn.at[ri_bufs[b]], bufs[b], sem, add=True)
            for b in range(_KB)
        ]
        for d in descs:
            d.wait()

        for b in range(_KB):
            def relurow(r, carry2, _b=b):
                for cc in range(_HID // 16):
                    sl = pl.ds(cc * 16, 16)
                    bufs[_b][r, sl] = jnp.maximum(bufs[_b][r, sl], 0.0)
                return carry2

            lax.fori_loop(0, _GB, relurow, 0)

        descs = [
            pltpu.async_copy(bufs[b], acc.at[di_bufs[b]], sem, add=True)
            for b in range(_KB)
        ]
        for d in descs:
            d.wait()
        return carry

    lax.fori_loop(0, _NB1 // _KB, burst, 0)
    plsc.subcore_barrier()

    pltpu.sync_copy(acc.at[pl.ds(srow, _NROWS)],
                    out.at[cid, pl.ds(srow, _NROWS)])


def _sc_fin(h0b, a, gn, src1, rev1, dst1, zeros):
    f = pl.kernel(
        _sc_fin_body,
        out_type=jax.ShapeDtypeStruct((_NC, _NP, _HID), jnp.float32),
        mesh=_sc_mesh(),
        scratch_types=[
            [pltpu.VMEM((_GB,), jnp.int32) for _ in range(_KB)],
            [pltpu.VMEM((_GB,), jnp.int32) for _ in range(_KB)],
            [pltpu.VMEM((_GB,), jnp.int32) for _ in range(_KB)],
            [pltpu.VMEM((_GB, _HID), jnp.float32) for _ in range(_KB)],
            pltpu.VMEM_SHARED((_NP, _HID), jnp.float32),
            pltpu.SemaphoreType.DMA,
            pltpu.SemaphoreType.DMA,
        ],
    )
    return f(h0b, a, gn, src1, rev1, dst1, zeros)


def _combine_neg_body(p_ref, o_ref):
    o_ref[...] = -(p_ref[0] + p_ref[1])


def _combine_neg(p):
    return pl.pallas_call(
        _combine_neg_body,
        out_shape=jax.ShapeDtypeStruct((_NP, _HID), jnp.float32),
    )(p)


def _node_mm_body(x_ref, w_ref, o_ref):
    o_ref[...] = jnp.dot(x_ref[...], w_ref[...],
                         preferred_element_type=jnp.float32)


def _node_matmul(x, w):
    n, d = x.shape
    return pl.pallas_call(
        _node_mm_body,
        out_shape=jax.ShapeDtypeStruct((n, w.shape[1]), jnp.float32),
    )(x, w)


def _layer0_body(xs_ref, ea_ref, wie_ref, bi_ref, bh_ref, whn_ref,
                 h0b_ref, g0n_ref):
    h0 = xs_ref[...] + jnp.dot(ea_ref[...], wie_ref[...],
                               preferred_element_type=jnp.float32) + bi_ref[...]
    h0b_ref[...] = h0 + bh_ref[...]
    g0n_ref[...] = jnp.dot(jnp.maximum(h0, 0.0), whn_ref[...],
                           preferred_element_type=jnp.float32)


def _layer0(xwi_src, edge_attr, wie, bi, bh, whn):
    grid = (_E // _BE,)
    return pl.pallas_call(
        _layer0_body,
        grid=grid,
        in_specs=[
            pl.BlockSpec((_BE, _D), lambda i: (i, 0)),
            pl.BlockSpec((_BE, _DE), lambda i: (i, 0)),
            pl.BlockSpec((_DE, _HID), lambda i: (0, 0)),
            pl.BlockSpec((1, _HID), lambda i: (0, 0)),
            pl.BlockSpec((1, _HID), lambda i: (0, 0)),
            pl.BlockSpec((_HID, _HID), lambda i: (0, 0)),
        ],
        out_specs=[
            pl.BlockSpec((_BE, _HID), lambda i: (i, 0)),
            pl.BlockSpec((_BE, _HID), lambda i: (i, 0)),
        ],
        out_shape=[
            jax.ShapeDtypeStruct((_EP, _HID), jnp.float32),
            jax.ShapeDtypeStruct((_EP, _HID), jnp.float32),
        ],
        compiler_params=pltpu.CompilerParams(
            dimension_semantics=("parallel",)),
    )(xwi_src, edge_attr, wie, bi, bh, whn)


def _layer_mid_body(h0b_ref, m_ref, whn_ref, g_ref):
    ht = jnp.maximum(h0b_ref[...] + m_ref[...], 0.0)
    g_ref[...] = jnp.dot(ht, whn_ref[...], preferred_element_type=jnp.float32)


def _layer_mid(h0b, m, whn):
    grid = (_E // _BE,)
    return pl.pallas_call(
        _layer_mid_body,
        grid=grid,
        in_specs=[
            pl.BlockSpec((_BE, _HID), lambda i: (i, 0)),
            pl.BlockSpec((_BE, _HID), lambda i: (i, 0)),
            pl.BlockSpec((_HID, _HID), lambda i: (0, 0)),
        ],
        out_specs=pl.BlockSpec((_BE, _HID), lambda i: (i, 0)),
        out_shape=jax.ShapeDtypeStruct((_EP, _HID), jnp.float32),
        compiler_params=pltpu.CompilerParams(
            dimension_semantics=("parallel",)),
    )(h0b, m, whn)


def _layer_last_body(h0b_ref, m_ref, ht_ref):
    ht_ref[...] = jnp.maximum(h0b_ref[...] + m_ref[...], 0.0)


def _layer_last(h0b, m):
    grid = (_E // _BE,)
    return pl.pallas_call(
        _layer_last_body,
        grid=grid,
        in_specs=[
            pl.BlockSpec((_BE, _HID), lambda i: (i, 0)),
            pl.BlockSpec((_BE, _HID), lambda i: (i, 0)),
        ],
        out_specs=pl.BlockSpec((_BE, _HID), lambda i: (i, 0)),
        out_shape=jax.ShapeDtypeStruct((_EP, _HID), jnp.float32),
        compiler_params=pltpu.CompilerParams(
            dimension_semantics=("parallel",)),
    )(h0b, m)


def _final_body(x_ref, a_ref, wox_ref, wom_ref, bo_ref, o_ref):
    a = a_ref[0, :_N] + a_ref[1, :_N]
    rs = jnp.sum(a, axis=1, keepdims=True)
    m = jnp.where(rs == 0.0, x_ref[...], a)
    o = (jnp.dot(x_ref[...], wox_ref[...], preferred_element_type=jnp.float32)
         + jnp.dot(m, wom_ref[...], preferred_element_type=jnp.float32)
         + bo_ref[...])
    o_ref[...] = jnp.maximum(o, 0.0)


def _final(x, a2, wox, wom, bo):
    return pl.pallas_call(
        _final_body,
        out_shape=jax.ShapeDtypeStruct((_N, _HID), jnp.float32),
    )(x, a2, wox, wom, bo.reshape(1, _HID))


def kernel(x, edge_index, edge_attr, rev_edge_index, Wi, bi, Wh, bh, Wo, bo):
    pad = jnp.arange(_PADE, dtype=jnp.int32)
    src1 = jnp.concatenate([edge_index[0], pad % _N])
    dst1 = jnp.concatenate([edge_index[1], _N + pad % (_NP - _N)])
    rev1 = jnp.concatenate([rev_edge_index, pad % _E])
    src2 = src1.reshape(_NBLK, _GB)
    rev2 = rev1.reshape(_NBLK, _GB)

    zeros = jnp.zeros((_NROWS, _HID), jnp.float32)
    whn = -Wh

    xwi = _node_matmul(x, Wi[:_D])
    xwi_src = _sc_gather(xwi, src2)
    h0b, g0n = _layer0(xwi_src, edge_attr, Wi[_D:], bi.reshape(1, _HID),
                       bh.reshape(1, _HID), whn)

    a0 = _combine_neg(_sc_scatter(g0n, dst1, zeros))
    m1 = _sc_msg(a0, g0n, src2, rev2)
    g1n = _layer_mid(h0b, m1, whn)

    a1 = _combine_neg(_sc_scatter(g1n, dst1, zeros))
    a2p = _sc_fin(h0b, a1, g1n, src1, rev1, dst1, zeros)
    return _final(x, a2p, Wo[:_D], Wo[_D:], bo)


# speedup vs baseline: 3.8488x; 1.0168x over previous
import functools

import jax
import jax.numpy as jnp
from jax import lax
from jax.experimental import pallas as pl
from jax.experimental.pallas import tpu as pltpu
from jax.experimental.pallas import tpu_sc as plsc

_N = 10000
_E = 320000
_D = 128
_DE = 16
_HID = 128
_HC = _HID // 2

_BE = 2560

_NC = 2
_NS = 16
_NW = _NC * _NS
_GB = 80
_KB = 4
_KG = 8
_EP = 327680
_PADE = _EP - _E
_NBLK = _EP // _GB
_RW = _NBLK // _NW
_RW1 = _NBLK // _NS
_NP = 10240
_NROWS = _NP // _NS
_DR = 16
_DCH = _NROWS // _DR


def _sc_mesh(num_cores=_NC):
    return plsc.VectorSubcoreMesh(core_axis_name="c", subcore_axis_name="s",
                                  num_cores=num_cores)


def _sc_gather_body(table, idx2, out, idx_v, bufs, sem):
    wid = lax.axis_index("s") * _NC + lax.axis_index("c")
    rbase = wid * _RW
    pltpu.sync_copy(idx2.at[pl.ds(rbase, _RW)], idx_v)

    def burst(g, carry):
        descs = [
            pltpu.async_copy(table.at[idx_v.at[g * _KG + b]], bufs[b], sem)
            for b in range(_KG)
        ]
        for d in descs:
            d.wait()
        obase = (rbase + g * _KG) * _GB
        descs = [
            pltpu.async_copy(bufs[b], out.at[pl.ds(obase + b * _GB, _GB)], sem)
            for b in range(_KG)
        ]
        for d in descs:
            d.wait()
        return carry

    lax.fori_loop(0, _RW // _KG, burst, 0)


def _sc_gather(table, idx2):
    f = pl.kernel(
        _sc_gather_body,
        out_type=jax.ShapeDtypeStruct((_EP, _HID), jnp.float32),
        mesh=_sc_mesh(),
        scratch_types=[
            pltpu.VMEM((_RW, _GB), jnp.int32),
            [pltpu.VMEM((_GB, _HID), jnp.float32) for _ in range(_KG)],
            pltpu.SemaphoreType.DMA,
        ],
    )
    return f(table, idx2)


def _sc_msg_body(a, gn, src2, rev2, out, si_v, ri_v, bufs, sem):
    wid = lax.axis_index("s") * _NC + lax.axis_index("c")
    rbase = wid * _RW
    pltpu.sync_copy(src2.at[pl.ds(rbase, _RW)], si_v)
    pltpu.sync_copy(rev2.at[pl.ds(rbase, _RW)], ri_v)

    def burst(g, carry):
        descs = [
            pltpu.async_copy(a.at[si_v.at[g * _KG + b]], bufs[b], sem)
            for b in range(_KG)
        ]
        for d in descs:
            d.wait()
        descs = [
            pltpu.async_copy(gn.at[ri_v.at[g * _KG + b]], bufs[b], sem,
                             add=True)
            for b in range(_KG)
        ]
        for d in descs:
            d.wait()
        obase = (rbase + g * _KG) * _GB
        descs = [
            pltpu.async_copy(bufs[b], out.at[pl.ds(obase + b * _GB, _GB)], sem)
            for b in range(_KG)
        ]
        for d in descs:
            d.wait()
        return carry

    lax.fori_loop(0, _RW // _KG, burst, 0)


def _sc_msg(a, gn, src2, rev2):
    f = pl.kernel(
        _sc_msg_body,
        out_type=jax.ShapeDtypeStruct((_EP, _HID), jnp.float32),
        mesh=_sc_mesh(),
        scratch_types=[
            pltpu.VMEM((_RW, _GB), jnp.int32),
            pltpu.VMEM((_RW, _GB), jnp.int32),
            [pltpu.VMEM((_GB, _HID), jnp.float32) for _ in range(_KG)],
            pltpu.SemaphoreType.DMA,
        ],
    )
    return f(a, gn, src2, rev2)


_PW1 = _EP // _NW
_NB1 = _PW1 // _GB


def _sc_scatter_body(vals, dst1, zeros, out, idx_bufs, bufs, acc, sem, isem):
    cid = lax.axis_index("c")
    sid = lax.axis_index("s")

    srow = sid * _NROWS
    pltpu.sync_copy(zeros.at[pl.ds(0, _NROWS)], acc.at[pl.ds(srow, _NROWS)])
    plsc.subcore_barrier()

    base = cid * (_EP // _NC) + sid * _PW1

    def burst(g, carry):
        obase = base + g * _KB * _GB
        idescs = [
            pltpu.async_copy(dst1.at[pl.ds(obase + b * _GB, _GB)],
                             idx_bufs[b], isem)
            for b in range(_KB)
        ]
        vdescs = [
            pltpu.async_copy(vals.at[pl.ds(obase + b * _GB, _GB)], bufs[b],
                             sem)
            for b in range(_KB)
        ]
        for d in idescs:
            d.wait()
        for d in vdescs:
            d.wait()
        descs = [
            pltpu.async_copy(bufs[b], acc.at[idx_bufs[b]], sem, add=True)
            for b in range(_KB)
        ]
        for d in descs:
            d.wait()
        return carry

    lax.fori_loop(0, _NB1 // _KB, burst, 0)
    plsc.subcore_barrier()

    pltpu.sync_copy(acc.at[pl.ds(srow, _NROWS)],
                    out.at[cid, pl.ds(srow, _NROWS)])


def _sc_scatter(vals, dst1, zeros):
    f = pl.kernel(
        _sc_scatter_body,
        out_type=jax.ShapeDtypeStruct((_NC, _NP, _HID), jnp.float32),
        mesh=_sc_mesh(),
        scratch_types=[
            [pltpu.VMEM((_GB,), jnp.int32) for _ in range(_KB)],
            [pltpu.VMEM((_GB, _HID), jnp.float32) for _ in range(_KB)],
            pltpu.VMEM_SHARED((_NP, _HID), jnp.float32),
            pltpu.SemaphoreType.DMA,
            pltpu.SemaphoreType.DMA,
        ],
    )
    return f(vals, dst1, zeros)


def _sc_fin_body(h0b, a, gn, src1, rev1, dst1, zeros, out,
                 si_bufs, ri_bufs, di_bufs, bufs, acc, sem, isem):
    cid = lax.axis_index("c")
    sid = lax.axis_index("s")

    srow = sid * _NROWS
    pltpu.sync_copy(zeros.at[pl.ds(0, _NROWS)], acc.at[pl.ds(srow, _NROWS)])
    plsc.subcore_barrier()

    base = cid * (_EP // _NC) + sid * _PW1

    def burst(g, carry):
        obase = base + g * _KB * _GB
        idescs = []
        for b in range(_KB):
            off = pl.ds(obase + b * _GB, _GB)
            idescs.append(pltpu.async_copy(src1.at[off], si_bufs[b], isem))
            idescs.append(pltpu.async_copy(rev1.at[off], ri_bufs[b], isem))
            idescs.append(pltpu.async_copy(dst1.at[off], di_bufs[b], isem))
        hdescs = [
            pltpu.async_copy(h0b.at[pl.ds(obase + b * _GB, _GB)], bufs[b],
                             sem)
            for b in range(_KB)
        ]
        for d in idescs:
            d.wait()
        for d in hdescs:
            d.wait()
        descs = [
            pltpu.async_copy(a.at[si_bufs[b]], bufs[b], sem, add=True)
            for b in range(_KB)
        ]
        for d in descs:
            d.wait()
        descs = [
            pltpu.async_copy(gn.at[ri_bufs[b]], bufs[b], sem, add=True)
            for b in range(_KB)
        ]
        for d in descs:
            d.wait()

        for b in range(_KB):
            def relurow(r, carry2, _b=b):
                for cc in range(_HID // 16):
                    sl = pl.ds(cc * 16, 16)
                    bufs[_b][r, sl] = jnp.maximum(bufs[_b][r, sl], 0.0)
                return carry2

            lax.fori_loop(0, _GB, relurow, 0)

        descs = [
            pltpu.async_copy(bufs[b], acc.at[di_bufs[b]], sem, add=True)
            for b in range(_KB)
        ]
        for d in descs:
            d.wait()
        return carry

    lax.fori_loop(0, _NB1 // _KB, burst, 0)
    plsc.subcore_barrier()

    pltpu.sync_copy(acc.at[pl.ds(srow, _NROWS)],
                    out.at[cid, pl.ds(srow, _NROWS)])


def _sc_fin(h0b, a, gn, src1, rev1, dst1, zeros):
    f = pl.kernel(
        _sc_fin_body,
        out_type=jax.ShapeDtypeStruct((_NC, _NP, _HID), jnp.float32),
        mesh=_sc_mesh(),
        scratch_types=[
            [pltpu.VMEM((_GB,), jnp.int32) for _ in range(_KB)],
            [pltpu.VMEM((_GB,), jnp.int32) for _ in range(_KB)],
            [pltpu.VMEM((_GB,), jnp.int32) for _ in range(_KB)],
            [pltpu.VMEM((_GB, _HID), jnp.float32) for _ in range(_KB)],
            pltpu.VMEM_SHARED((_NP, _HID), jnp.float32),
            pltpu.SemaphoreType.DMA,
            pltpu.SemaphoreType.DMA,
        ],
    )
    return f(h0b, a, gn, src1, rev1, dst1, zeros)


def _combine_neg_body(p_ref, o_ref):
    o_ref[...] = -(p_ref[0] + p_ref[1])


def _combine_neg(p):
    return pl.pallas_call(
        _combine_neg_body,
        out_shape=jax.ShapeDtypeStruct((_NP, _HID), jnp.float32),
    )(p)


def _node_mm_body(x_ref, w_ref, o_ref):
    o_ref[...] = jnp.dot(x_ref[...], w_ref[...],
                         preferred_element_type=jnp.float32)


def _node_matmul(x, w):
    n, d = x.shape
    return pl.pallas_call(
        _node_mm_body,
        out_shape=jax.ShapeDtypeStruct((n, w.shape[1]), jnp.float32),
    )(x, w)


def _layer0_body(xs_ref, ea_ref, wie_ref, bi_ref, bh_ref, whn_ref,
                 h0b_ref, g0n_ref):
    h0 = xs_ref[...] + jnp.dot(ea_ref[...], wie_ref[...],
                               preferred_element_type=jnp.float32) + bi_ref[...]
    h0b_ref[...] = h0 + bh_ref[...]
    g0n_ref[...] = jnp.dot(jnp.maximum(h0, 0.0), whn_ref[...],
                           preferred_element_type=jnp.float32)


def _layer0(xwi_src, edge_attr, wie, bi, bh, whn):
    grid = (_E // _BE,)
    return pl.pallas_call(
        _layer0_body,
        grid=grid,
        in_specs=[
            pl.BlockSpec((_BE, _D), lambda i: (i, 0)),
            pl.BlockSpec((_BE, _DE), lambda i: (i, 0)),
            pl.BlockSpec((_DE, _HID), lambda i: (0, 0)),
            pl.BlockSpec((1, _HID), lambda i: (0, 0)),
            pl.BlockSpec((1, _HID), lambda i: (0, 0)),
            pl.BlockSpec((_HID, _HID), lambda i: (0, 0)),
        ],
        out_specs=[
            pl.BlockSpec((_BE, _HID), lambda i: (i, 0)),
            pl.BlockSpec((_BE, _HID), lambda i: (i, 0)),
        ],
        out_shape=[
            jax.ShapeDtypeStruct((_EP, _HID), jnp.float32),
            jax.ShapeDtypeStruct((_EP, _HID), jnp.float32),
        ],
        compiler_params=pltpu.CompilerParams(
            dimension_semantics=("parallel",)),
    )(xwi_src, edge_attr, wie, bi, bh, whn)


def _layer_mid_body(h0b_ref, m_ref, whn_ref, g_ref):
    ht = jnp.maximum(h0b_ref[...] + m_ref[...], 0.0)
    g_ref[...] = jnp.dot(ht, whn_ref[...], preferred_element_type=jnp.float32)


def _layer_mid(h0b, m, whn):
    grid = (_E // _BE,)
    return pl.pallas_call(
        _layer_mid_body,
        grid=grid,
        in_specs=[
            pl.BlockSpec((_BE, _HID), lambda i: (i, 0)),
            pl.BlockSpec((_BE, _HID), lambda i: (i, 0)),
            pl.BlockSpec((_HID, _HID), lambda i: (0, 0)),
        ],
        out_specs=pl.BlockSpec((_BE, _HID), lambda i: (i, 0)),
        out_shape=jax.ShapeDtypeStruct((_EP, _HID), jnp.float32),
        compiler_params=pltpu.CompilerParams(
            dimension_semantics=("parallel",)),
    )(h0b, m, whn)


def _layer_last_body(h0b_ref, m_ref, ht_ref):
    ht_ref[...] = jnp.maximum(h0b_ref[...] + m_ref[...], 0.0)


def _layer_last(h0b, m):
    grid = (_E // _BE,)
    return pl.pallas_call(
        _layer_last_body,
        grid=grid,
        in_specs=[
            pl.BlockSpec((_BE, _HID), lambda i: (i, 0)),
            pl.BlockSpec((_BE, _HID), lambda i: (i, 0)),
        ],
        out_specs=pl.BlockSpec((_BE, _HID), lambda i: (i, 0)),
        out_shape=jax.ShapeDtypeStruct((_EP, _HID), jnp.float32),
        compiler_params=pltpu.CompilerParams(
            dimension_semantics=("parallel",)),
    )(h0b, m)


def _final_body(x_ref, a_ref, wox_ref, wom_ref, bo_ref, o_ref):
    a = a_ref[0, :_N] + a_ref[1, :_N]
    rs = jnp.sum(a, axis=1, keepdims=True)
    m = jnp.where(rs == 0.0, x_ref[...], a)
    o = (jnp.dot(x_ref[...], wox_ref[...], preferred_element_type=jnp.float32)
         + jnp.dot(m, wom_ref[...], preferred_element_type=jnp.float32)
         + bo_ref[...])
    o_ref[...] = jnp.maximum(o, 0.0)


def _final(x, a2, wox, wom, bo):
    return pl.pallas_call(
        _final_body,
        out_shape=jax.ShapeDtypeStruct((_N, _HID), jnp.float32),
    )(x, a2, wox, wom, bo.reshape(1, _HID))


def kernel(x, edge_index, edge_attr, rev_edge_index, Wi, bi, Wh, bh, Wo, bo):
    pad = jnp.arange(_PADE, dtype=jnp.int32)
    src1 = jnp.concatenate([edge_index[0], pad % _N])
    dst1 = jnp.concatenate([edge_index[1], _N + pad % (_NP - _N)])
    rev1 = jnp.concatenate([rev_edge_index, pad % _E])
    src2 = src1.reshape(_NBLK, _GB)
    rev2 = rev1.reshape(_NBLK, _GB)

    zeros = jnp.zeros((_NROWS, _HID), jnp.float32)
    whn = -Wh

    xwi = _node_matmul(x, Wi[:_D])
    xwi_src = _sc_gather(xwi, src2)
    h0b, g0n = _layer0(xwi_src, edge_attr, Wi[_D:], bi.reshape(1, _HID),
                       bh.reshape(1, _HID), whn)

    a0 = _combine_neg(_sc_scatter(g0n, dst1, zeros))
    m1 = _sc_msg(a0, g0n, src2, rev2)
    g1n = _layer_mid(h0b, m1, whn)

    a1 = _combine_neg(_sc_scatter(g1n, dst1, zeros))
    a2p = _sc_fin(h0b, a1, g1n, src1, rev1, dst1, zeros)
    return _final(x, a2p, Wo[:_D], Wo[_D:], bo)
